# baseline (speedup 1.0000x reference)
import jax
import jax.numpy as jnp
from jax.experimental import pallas as pl
from jax.experimental.pallas import tpu as pltpu

C = 1000
B = 16384
BB = 512
NB = B // BB


def _cm_body(pred_ref, tgt_ref, out_ref):
    i = pl.program_id(0)
    x = pred_ref[...]
    mx = jnp.max(x, axis=1, keepdims=True)
    col = jax.lax.broadcasted_iota(jnp.int32, x.shape, 1)
    p = jnp.min(jnp.where(x == mx, col, C), axis=1)
    t = tgt_ref[0, 0, :]

    rows = jax.lax.broadcasted_iota(jnp.int32, (C, BB), 0)
    oh_t = (rows == t[None, :]).astype(jnp.bfloat16)
    cols = jax.lax.broadcasted_iota(jnp.int32, (BB, C), 1)
    oh_p = (cols == p[:, None]).astype(jnp.bfloat16)
    part = jnp.dot(oh_t, oh_p, preferred_element_type=jnp.float32)

    @pl.when(i == 0)
    def _():
        out_ref[...] = part

    @pl.when(i != 0)
    def _():
        out_ref[...] += part


def kernel(prediction, target):
    tgt3 = target.reshape(NB, 1, BB)
    return pl.pallas_call(
        _cm_body,
        grid=(NB,),
        in_specs=[
            pl.BlockSpec((BB, C), lambda i: (i, 0)),
            pl.BlockSpec((1, 1, BB), lambda i: (i, 0, 0)),
        ],
        out_specs=pl.BlockSpec((C, C), lambda i: (0, 0)),
        out_shape=jax.ShapeDtypeStruct((C, C), jnp.float32),
        compiler_params=pltpu.CompilerParams(
            dimension_semantics=("arbitrary",),
        ),
    )(prediction, tgt3)
